# baseline (speedup 1.0000x reference)
import functools

import jax
import jax.numpy as jnp
from jax import lax
from jax.experimental import pallas as pl
from jax.experimental.pallas import tpu as pltpu
from jax.experimental.pallas import tpu_sc as plsc

N_FIELDS = 26
VOCAB = 100000
EMB = 16
EPS = 1e-5

NW = 32
NROWS = N_FIELDS * EMB
RPW = NROWS // NW


def _gather_kernel(B):
    mesh = plsc.VectorSubcoreMesh(core_axis_name="c", subcore_axis_name="s")

    @functools.partial(
        pl.kernel,
        out_type=jax.ShapeDtypeStruct((NROWS, B), jnp.float32),
        mesh=mesh,
        compiler_params=pltpu.CompilerParams(use_tc_tiling_on_sc=True,
                                             needs_layout_passes=False),
        scratch_types=[
            pltpu.VMEM((VOCAB,), jnp.float32),
            pltpu.VMEM((B // NW * NW,), jnp.float32),
            pltpu.SemaphoreType.DMA,
            pltpu.SemaphoreType.DMA,
            pltpu.SemaphoreType.DMA,
            pltpu.SemaphoreType.DMA,
        ],
    )
    def gk(tt_hbm, idxf_hbm, out_hbm, row_v, io_v, rsem, isem, wsem, wsem2):
        wid = lax.axis_index("s") * 2 + lax.axis_index("c")
        row0 = wid * RPW

        def row_dma(k):
            return pltpu.async_copy(tt_hbm.at[row0 + k], row_v, rsem)

        def idx_dma(k):
            f = lax.shift_right_logical(row0 + k, 4)
            return pltpu.async_copy(idxf_hbm.at[f], io_v, isem)

        cr = row_dma(0)
        ci = idx_dma(0)
        for k in range(RPW):
            cr.wait()
            ci.wait()

            def blk(kk, carry):
                for j in range(16):
                    off = kk * 256 + j * 16
                    iv = plsc.bitcast(io_v[pl.ds(off, 16)], jnp.int32)
                    io_v[pl.ds(off, 16)] = plsc.load_gather(row_v, [iv])
                return carry

            half = B // 512
            lax.fori_loop(0, half, blk, 0)
            wb0 = pltpu.async_copy(
                io_v.at[pl.ds(0, B // 2)],
                out_hbm.at[row0 + k, pl.ds(0, B // 2)], wsem)
            lax.fori_loop(half, 2 * half, blk, 0)
            if k + 1 < RPW:
                cr = row_dma(k + 1)
            wb1 = pltpu.async_copy(
                io_v.at[pl.ds(B // 2, B // 2)],
                out_hbm.at[row0 + k, pl.ds(B // 2, B // 2)], wsem2)
            wb0.wait()
            wb1.wait()
            if k + 1 < RPW:
                ci = idx_dma(k + 1)

    return gk


def _mlp_body(xn_ref, emb_ref, w0n_ref, w0e_ref, b0_ref, w1_ref, b1_ref,
              w2_ref, b2_ref, w3_ref, b3_ref, out_ref):
    dg = lambda a, b, ca, cb: lax.dot_general(
        a, b, (((ca,), (cb,)), ((), ())),
        preferred_element_type=jnp.float32)
    h = dg(w0e_ref[...], emb_ref[...], 0, 0)
    h = h + dg(w0n_ref[...], xn_ref[...], 0, 1)
    h = jnp.maximum(h + b0_ref[...], 0.0)
    h = jnp.maximum(dg(w1_ref[...], h, 0, 0) + b1_ref[...], 0.0)
    h = jnp.maximum(dg(w2_ref[...], h, 0, 0) + b2_ref[...], 0.0)
    out_ref[...] = dg(w3_ref[...], h, 0, 0) + b3_ref[...]


def kernel(x_num, x_cat, tables, W0, b0, g0, be0, W1, b1, g1, be1,
           W2, b2, g2, be2, W3, b3):
    B = x_num.shape[0]

    tt = tables.transpose(0, 2, 1).reshape(NROWS, VOCAB)
    idxf = lax.bitcast_convert_type(x_cat.astype(jnp.int32).T, jnp.float32)

    inv = 1.0 / jnp.sqrt(1.0 + EPS)
    s0, s1, s2 = g0 * inv, g1 * inv, g2 * inv
    W0f = W0 * s0[None, :]
    b0T = (b0 * s0 + be0)[:, None]
    W1f = W1 * s1[None, :]
    b1T = (b1 * s1 + be1)[:, None]
    W2f = W2 * s2[None, :]
    b2T = (b2 * s2 + be2)[:, None]
    W0n = W0f[:13]
    W0e = W0f[13:]
    w3p = jnp.pad(W3, ((0, 0), (0, 7)))
    b3T = jnp.pad(b3.reshape(1, 1), ((0, 7), (0, 0)))

    embT = _gather_kernel(B)(tt, idxf)

    BM = 4096
    nb = B // BM
    full = lambda s: pl.BlockSpec(s, lambda i: (0, 0))
    out2 = pl.pallas_call(
        _mlp_body,
        grid=(nb,),
        in_specs=[
            pl.BlockSpec((BM, 13), lambda i: (i, 0)),
            pl.BlockSpec((NROWS, BM), lambda i: (0, i)),
            full((13, 128)), full((NROWS, 128)), full((128, 1)),
            full((128, 64)), full((64, 1)),
            full((64, 32)), full((32, 1)),
            full((32, 8)), full((8, 1)),
        ],
        out_specs=pl.BlockSpec((8, BM), lambda i: (0, i)),
        out_shape=jax.ShapeDtypeStruct((8, B), jnp.float32),
    )(x_num, embT, W0n, W0e, b0T, W1f, b1T, W2f, b2T, w3p, b3T)

    return out2[0, :]


# speedup vs baseline: 1.0062x; 1.0062x over previous
import functools

import jax
import jax.numpy as jnp
from jax import lax
from jax.experimental import pallas as pl
from jax.experimental.pallas import tpu as pltpu
from jax.experimental.pallas import tpu_sc as plsc

N_FIELDS = 26
VOCAB = 100000
EMB = 16
EPS = 1e-5

NW = 32
NROWS = N_FIELDS * EMB
RPW = NROWS // NW


def _gather_kernel(B):
    mesh = plsc.VectorSubcoreMesh(core_axis_name="c", subcore_axis_name="s")

    @functools.partial(
        pl.kernel,
        out_type=jax.ShapeDtypeStruct((NROWS, B), jnp.float32),
        mesh=mesh,
        compiler_params=pltpu.CompilerParams(use_tc_tiling_on_sc=True,
                                             needs_layout_passes=False),
        scratch_types=[
            pltpu.VMEM((VOCAB,), jnp.float32),
            pltpu.VMEM((B // NW * NW,), jnp.float32),
            pltpu.SemaphoreType.DMA,
            pltpu.SemaphoreType.DMA,
            pltpu.SemaphoreType.DMA,
        ],
    )
    def gk(tt_hbm, idxf_hbm, out_hbm, row_v, io_v, rsem, isem, wsem):
        wid = lax.axis_index("s") * 2 + lax.axis_index("c")
        row0 = wid * RPW

        def row_dma(k):
            return pltpu.async_copy(tt_hbm.at[row0 + k], row_v, rsem)

        def idx_dma(k):
            f = lax.shift_right_logical(row0 + k, 4)
            return pltpu.async_copy(idxf_hbm.at[f], io_v, isem)

        cr = row_dma(0)
        ci = idx_dma(0)
        for k in range(RPW):
            cr.wait()
            ci.wait()

            def blk(kk, carry):
                for j in range(16):
                    off = kk * 256 + j * 16
                    iv = plsc.bitcast(io_v[pl.ds(off, 16)], jnp.int32)
                    io_v[pl.ds(off, 16)] = plsc.load_gather(row_v, [iv])
                return carry

            lax.fori_loop(0, B // 256, blk, 0)
            if k + 1 < RPW:
                cr = row_dma(k + 1)
            wb = pltpu.async_copy(io_v, out_hbm.at[row0 + k], wsem)
            wb.wait()
            if k + 1 < RPW:
                ci = idx_dma(k + 1)

    return gk


def _mlp_body(xn_ref, emb_ref, w0n_ref, w0e_ref, b0_ref, w1_ref, b1_ref,
              w2_ref, b2_ref, w3_ref, b3_ref, out_ref):
    dg = lambda a, b, ca, cb: lax.dot_general(
        a, b, (((ca,), (cb,)), ((), ())),
        preferred_element_type=jnp.float32)
    h = dg(w0e_ref[...], emb_ref[...], 0, 0)
    h = h + dg(w0n_ref[...], xn_ref[...], 0, 1)
    h = jnp.maximum(h + b0_ref[...], 0.0)
    h = jnp.maximum(dg(w1_ref[...], h, 0, 0) + b1_ref[...], 0.0)
    h = jnp.maximum(dg(w2_ref[...], h, 0, 0) + b2_ref[...], 0.0)
    out_ref[...] = dg(w3_ref[...], h, 0, 0) + b3_ref[...]


def kernel(x_num, x_cat, tables, W0, b0, g0, be0, W1, b1, g1, be1,
           W2, b2, g2, be2, W3, b3):
    B = x_num.shape[0]

    tt = tables.transpose(0, 2, 1).reshape(NROWS, VOCAB)
    idxf = lax.bitcast_convert_type(x_cat.astype(jnp.int32).T, jnp.float32)

    inv = 1.0 / jnp.sqrt(1.0 + EPS)
    s0, s1, s2 = g0 * inv, g1 * inv, g2 * inv
    W0f = W0 * s0[None, :]
    b0T = (b0 * s0 + be0)[:, None]
    W1f = W1 * s1[None, :]
    b1T = (b1 * s1 + be1)[:, None]
    W2f = W2 * s2[None, :]
    b2T = (b2 * s2 + be2)[:, None]
    W0n = W0f[:13]
    W0e = W0f[13:]
    w3p = jnp.pad(W3, ((0, 0), (0, 7)))
    b3T = jnp.pad(b3.reshape(1, 1), ((0, 7), (0, 0)))

    embT = _gather_kernel(B)(tt, idxf)

    BM = 4096
    nb = B // BM
    full = lambda s: pl.BlockSpec(s, lambda i: (0, 0))
    out2 = pl.pallas_call(
        _mlp_body,
        grid=(nb,),
        in_specs=[
            pl.BlockSpec((BM, 13), lambda i: (i, 0)),
            pl.BlockSpec((NROWS, BM), lambda i: (0, i)),
            full((13, 128)), full((NROWS, 128)), full((128, 1)),
            full((128, 64)), full((64, 1)),
            full((64, 32)), full((32, 1)),
            full((32, 8)), full((8, 1)),
        ],
        out_specs=pl.BlockSpec((8, BM), lambda i: (0, i)),
        out_shape=jax.ShapeDtypeStruct((8, B), jnp.float32),
    )(x_num, embT, W0n, W0e, b0T, W1f, b1T, W2f, b2T, w3p, b3T)

    return out2[0, :]
